# baseline (speedup 1.0000x reference)
import functools

import jax
import jax.numpy as jnp
from jax import lax
from jax.experimental import pallas as pl
from jax.experimental.pallas import tpu as pltpu
from jax.experimental.pallas import tpu_sc as plsc

N = 10000
E = 320000
D = 128
DE = 16
H = 128
L = 2

NC = 2
NS = 16
NW = NC * NS
CHUNK = 128
NCHUNK = 81
EPT = NCHUNK * CHUNK
E_PAD = NW * EPT
ROWS_PER_TILE = 632
ACC_ROWS = NS * ROWS_PER_TILE
WEP = 128
G3 = 3 * H

NBUF = 3
ROUNDS = NCHUNK // NBUF
assert ROUNDS * NBUF == NCHUNK

_sc_mesh = plsc.VectorSubcoreMesh(core_axis_name="c", subcore_axis_name="s")


def _run_pipeline(row_start, row_wait, idx_start, idx_wait,
                  dring, bufs, ssems, acc_sh):
    for b in range(NBUF):
        idx_start(b, b)
    for b in range(NBUF):
        idx_wait(b)
        row_start(b, b)

    def outer(cc, carry):
        for b in range(NBUF):
            row_wait(cc * NBUF + b, b)
            pltpu.async_copy(bufs[b], acc_sh.at[dring.at[b, 0]], ssems[b], add=True)

        @pl.when(cc < ROUNDS - 1)
        def _():
            for b in range(NBUF):
                pltpu.make_async_copy(bufs[b], acc_sh.at[dring.at[b, 0]], ssems[b]).wait()
                idx_start((cc + 1) * NBUF + b, b)
            for b in range(NBUF):
                idx_wait(b)
                row_start((cc + 1) * NBUF + b, b)
        return carry

    lax.fori_loop(0, ROUNDS, outer, None)
    for b in range(NBUF):
        pltpu.make_async_copy(bufs[b], acc_sh.at[dring.at[b, 0]], ssems[b]).wait()


@functools.partial(
    pl.kernel,
    mesh=_sc_mesh,
    out_type=jax.ShapeDtypeStruct((NC, ACC_ROWS, D), jnp.float32),
    scratch_types=[
        pltpu.VMEM((NBUF, 1, CHUNK), jnp.int32),
        pltpu.VMEM((NBUF, 1, CHUNK), jnp.int32),
        pltpu.VMEM_SHARED((ACC_ROWS, D), jnp.float32),
    ] + [pltpu.VMEM((CHUNK, D), jnp.float32)] * NBUF
      + [pltpu.SemaphoreType.DMA] * (4 * NBUF),
)
def _sc_gather_segsum(cur_hbm, src_hbm, dst_hbm, zeros_hbm, out_hbm,
                      sring, dring, acc_sh, *rest):
    bufs = rest[:NBUF]
    gsems = rest[NBUF:2 * NBUF]
    ssems = rest[2 * NBUF:3 * NBUF]
    isems = rest[3 * NBUF:4 * NBUF]
    jsems = rest[4 * NBUF:5 * NBUF]
    cid = lax.axis_index("c")
    sid = lax.axis_index("s")
    w = cid * NS + sid
    pltpu.sync_copy(zeros_hbm, acc_sh.at[pl.ds(sid * ROWS_PER_TILE, ROWS_PER_TILE)])
    plsc.subcore_barrier()

    def idx_start(c, b):
        pltpu.async_copy(src_hbm.at[w * NCHUNK + c], sring.at[b], isems[b])
        pltpu.async_copy(dst_hbm.at[w * NCHUNK + c], dring.at[b], jsems[b])

    def idx_wait(b):
        pltpu.make_async_copy(src_hbm.at[0], sring.at[b], isems[b]).wait()
        pltpu.make_async_copy(dst_hbm.at[0], dring.at[b], jsems[b]).wait()

    def row_start(c, b):
        pltpu.async_copy(cur_hbm.at[sring.at[b, 0]], bufs[b], gsems[b])

    def row_wait(c, b):
        pltpu.make_async_copy(cur_hbm.at[sring.at[b, 0]], bufs[b], gsems[b]).wait()

    _run_pipeline(row_start, row_wait, idx_start, idx_wait,
                  dring, bufs, ssems, acc_sh)
    plsc.subcore_barrier()
    pltpu.sync_copy(
        acc_sh.at[pl.ds(sid * ROWS_PER_TILE, ROWS_PER_TILE)],
        out_hbm.at[cid, pl.ds(sid * ROWS_PER_TILE, ROWS_PER_TILE)],
    )


@functools.partial(
    pl.kernel,
    mesh=_sc_mesh,
    out_type=jax.ShapeDtypeStruct((NC, ACC_ROWS, WEP), jnp.float32),
    scratch_types=[
        pltpu.VMEM((NBUF, 1, CHUNK), jnp.int32),
        pltpu.VMEM_SHARED((ACC_ROWS, WEP), jnp.float32),
    ] + [pltpu.VMEM((CHUNK, WEP), jnp.float32)] * NBUF
      + [pltpu.SemaphoreType.DMA] * (3 * NBUF),
)
def _sc_edge_segsum(payload_hbm, dst_hbm, zeros_hbm, out_hbm,
                    dring, acc_sh, *rest):
    bufs = rest[:NBUF]
    gsems = rest[NBUF:2 * NBUF]
    ssems = rest[2 * NBUF:3 * NBUF]
    jsems = rest[3 * NBUF:4 * NBUF]
    cid = lax.axis_index("c")
    sid = lax.axis_index("s")
    w = cid * NS + sid
    pltpu.sync_copy(zeros_hbm, acc_sh.at[pl.ds(sid * ROWS_PER_TILE, ROWS_PER_TILE)])
    plsc.subcore_barrier()

    def idx_start(c, b):
        pltpu.async_copy(dst_hbm.at[w * NCHUNK + c], dring.at[b], jsems[b])

    def idx_wait(b):
        pltpu.make_async_copy(dst_hbm.at[0], dring.at[b], jsems[b]).wait()

    def row_start(c, b):
        pltpu.async_copy(payload_hbm.at[pl.ds(w * EPT + c * CHUNK, CHUNK)],
                         bufs[b], gsems[b])

    def row_wait(c, b):
        pltpu.make_async_copy(payload_hbm.at[pl.ds(0, CHUNK)],
                              bufs[b], gsems[b]).wait()

    _run_pipeline(row_start, row_wait, idx_start, idx_wait,
                  dring, bufs, ssems, acc_sh)
    plsc.subcore_barrier()
    pltpu.sync_copy(
        acc_sh.at[pl.ds(sid * ROWS_PER_TILE, ROWS_PER_TILE)],
        out_hbm.at[cid, pl.ds(sid * ROWS_PER_TILE, ROWS_PER_TILE)],
    )


_BR = 2528


def _tc_dense_body(g0_ref, g1_ref, e0_ref, e1_ref, wn_ref, we_ref, out_ref):
    g = g0_ref[...] + g1_ref[...]
    e = e0_ref[...] + e1_ref[...]
    agg = lax.dot_general(g, wn_ref[...], (((1,), (0,)), ((), ())),
                          preferred_element_type=jnp.float32)
    agg += lax.dot_general(e, we_ref[...], (((1,), (0,)), ((), ())),
                           preferred_element_type=jnp.float32)
    agg = jnp.maximum(agg, 0.0)
    i = jax.nn.sigmoid(agg[:, :H])
    ct = jnp.tanh(agg[:, H:2 * H])
    o = jax.nn.sigmoid(agg[:, 2 * H:])
    out_ref[...] = o * jnp.tanh(i * ct)


def _tc_dense(g0, g1, e0, e1, wn3, we3p):
    return pl.pallas_call(
        _tc_dense_body,
        grid=(ACC_ROWS // _BR,),
        in_specs=[
            pl.BlockSpec((_BR, D), lambda i: (i, 0)),
            pl.BlockSpec((_BR, D), lambda i: (i, 0)),
            pl.BlockSpec((_BR, WEP), lambda i: (i, 0)),
            pl.BlockSpec((_BR, WEP), lambda i: (i, 0)),
            pl.BlockSpec((D, G3), lambda i: (0, 0)),
            pl.BlockSpec((WEP, G3), lambda i: (0, 0)),
        ],
        out_specs=pl.BlockSpec((_BR, D), lambda i: (i, 0)),
        out_shape=jax.ShapeDtypeStruct((ACC_ROWS, D), jnp.float32),
    )(g0, g1, e0, e1, wn3, we3p)


_GATES = (0, 2, 3)


def kernel(x, edge_index, edge_attr, Wn, bn, We, be):
    src = edge_index[0]
    dst = edge_index[1]
    pad = E_PAD - E
    srcp = jnp.concatenate([src, jnp.zeros((pad,), jnp.int32)])
    dstp = jnp.concatenate([dst, jnp.full((pad,), N, jnp.int32)])
    src3 = srcp.reshape(NW * NCHUNK, 1, CHUNK)
    dst3 = dstp.reshape(NW * NCHUNK, 1, CHUNK)
    payload = jnp.concatenate(
        [edge_attr,
         jnp.ones((E, 1), jnp.float32),
         jnp.zeros((E, WEP - DE - 1), jnp.float32)], axis=1)
    payload = jnp.concatenate(
        [payload, jnp.zeros((pad, WEP), jnp.float32)], axis=0)
    zeros_d = jnp.zeros((ROWS_PER_TILE, D), jnp.float32)
    zeros_e = jnp.zeros((ROWS_PER_TILE, WEP), jnp.float32)

    ea = _sc_edge_segsum(payload, dst3, zeros_e)

    def mk_w(l):
        wn3 = jnp.concatenate([Wn[l, g, :D, :] for g in _GATES], axis=1)
        wep = jnp.concatenate([We[l, g] for g in _GATES], axis=1)
        brow = jnp.concatenate([bn[l, g] + be[l, g] for g in _GATES])[None, :]
        we3p = jnp.concatenate(
            [wep, brow, jnp.zeros((WEP - DE - 1, G3), jnp.float32)], axis=0)
        return wn3, we3p

    cur = x
    for l in range(L):
        g = _sc_gather_segsum(cur, src3, dst3, zeros_d)
        wn3, we3p = mk_w(l)
        cur = _tc_dense(g[0], g[1], ea[0], ea[1], wn3, we3p)
    return cur[:N]
